# baseline (speedup 1.0000x reference)
import functools

import jax
import jax.numpy as jnp
from jax import lax
from jax.experimental import pallas as pl
from jax.experimental.pallas import tpu as pltpu
from jax.experimental.pallas import tpu_sc as plsc

N_NODES = 10000
N_EDGES = 320000
IN_FEATS = 128
HIDDEN = 128
NUM_CLASSES = 16

NC = 2
NS = 16
NW = NC * NS
EDGES_PER_W = N_EDGES // NW
ACC_ROWS = 10240
ROWS_PER_TILE = ACC_ROWS // NS
ZROWS = 32
ROW_BLK = 2000

CHUNK = 125
SUPER = 16
NBLK = 5
CHUNK_W = 500
NCHUNK_W = EDGES_PER_W // CHUNK_W

_MESH = plsc.VectorSubcoreMesh(core_axis_name="c", subcore_axis_name="s")
_SC_PARAMS = pltpu.CompilerParams(use_tc_tiling_on_sc=False)


def _zero2d(ref, rows, cols):
    def body(r, carry):
        for k in range(cols // 16):
            ref[r, pl.ds(k * 16, 16)] = jnp.zeros((16,), jnp.float32)
        return carry
    lax.fori_loop(0, rows, body, 0)


def _fill_ones(ref, rows, cols):
    def body(r, carry):
        for k in range(cols // 16):
            ref[r, pl.ds(k * 16, 16)] = jnp.ones((16,), jnp.float32)
        return carry
    lax.fori_loop(0, rows, body, 0)


@functools.partial(
    pl.kernel,
    out_type=jax.ShapeDtypeStruct((NC, 2, ACC_ROWS, 16), jnp.float32),
    mesh=_MESH,
    compiler_params=_SC_PARAMS,
    scratch_types=[
        pltpu.VMEM((NCHUNK_W, CHUNK_W), jnp.int32),
        pltpu.VMEM((NCHUNK_W, CHUNK_W), jnp.int32),
        pltpu.VMEM((CHUNK_W, 16), jnp.float32),
        pltpu.VMEM((ZROWS, 16), jnp.float32),
        pltpu.VMEM_SHARED((ACC_ROWS, 16), jnp.float32),
        pltpu.VMEM_SHARED((ACC_ROWS, 16), jnp.float32),
    ],
)
def _deg_kernel(src_hbm, dst_hbm, out_hbm, src_v, dst_v, ones_v, zero_v,
                acc_src, acc_dst):
    c = lax.axis_index("c")
    s = lax.axis_index("s")
    wid = s * NC + c

    pltpu.sync_copy(src_hbm.at[wid], src_v)
    pltpu.sync_copy(dst_hbm.at[wid], dst_v)
    _fill_ones(ones_v, CHUNK_W, 16)
    _zero2d(zero_v, ZROWS, 16)
    row0 = s * ROWS_PER_TILE
    for t in range(ROWS_PER_TILE // ZROWS):
        pltpu.sync_copy(zero_v, acc_src.at[pl.ds(row0 + t * ZROWS, ZROWS)])
        pltpu.sync_copy(zero_v, acc_dst.at[pl.ds(row0 + t * ZROWS, ZROWS)])
    plsc.subcore_barrier()

    for j in range(NCHUNK_W):
        pltpu.sync_copy(ones_v, acc_src.at[src_v.at[j]], add=True)
        pltpu.sync_copy(ones_v, acc_dst.at[dst_v.at[j]], add=True)
    plsc.subcore_barrier()

    pltpu.sync_copy(acc_src.at[pl.ds(row0, ROWS_PER_TILE)],
                    out_hbm.at[c, 0, pl.ds(row0, ROWS_PER_TILE)])
    pltpu.sync_copy(acc_dst.at[pl.ds(row0, ROWS_PER_TILE)],
                    out_hbm.at[c, 1, pl.ds(row0, ROWS_PER_TILE)])


def _make_agg(F, ch, sup, nblk):
    @functools.partial(
        pl.kernel,
        out_type=jax.ShapeDtypeStruct((NC, ACC_ROWS, F), jnp.float32),
        mesh=_MESH,
        compiler_params=_SC_PARAMS,
        scratch_types=[
            pltpu.VMEM((sup, ch), jnp.int32),
            pltpu.VMEM((sup, ch), jnp.int32),
            pltpu.VMEM((ch, F), jnp.float32),
            pltpu.VMEM((ch, F), jnp.float32),
            pltpu.VMEM((ZROWS, F), jnp.float32),
            pltpu.VMEM_SHARED((ACC_ROWS, F), jnp.float32),
            pltpu.SemaphoreType.DMA,
            pltpu.SemaphoreType.DMA,
        ],
    )
    def _agg_kernel(src_hbm, dst_hbm, h_hbm, out_hbm, src_v, dst_v,
                    buf_a, buf_b, zero_v, acc, sem_a, sem_b):
        c = lax.axis_index("c")
        s = lax.axis_index("s")
        wid = s * NC + c

        _zero2d(zero_v, ZROWS, F)
        row0 = s * ROWS_PER_TILE
        for t in range(ROWS_PER_TILE // ZROWS):
            pltpu.sync_copy(zero_v, acc.at[pl.ds(row0 + t * ZROWS, ZROWS)])
        plsc.subcore_barrier()

        bufs = (buf_a, buf_b)
        sems = (sem_a, sem_b)
        for b in range(nblk):
            pltpu.sync_copy(src_hbm.at[wid, b], src_v)
            pltpu.sync_copy(dst_hbm.at[wid, b], dst_v)
            descs = [None, None]
            descs[0] = pltpu.async_copy(h_hbm.at[src_v.at[0]], bufs[0],
                                        sems[0])
            for j in range(sup):
                if j + 1 < sup:
                    descs[(j + 1) % 2] = pltpu.async_copy(
                        h_hbm.at[src_v.at[j + 1]], bufs[(j + 1) % 2],
                        sems[(j + 1) % 2])
                descs[j % 2].wait()
                pltpu.sync_copy(bufs[j % 2], acc.at[dst_v.at[j]], add=True)
        plsc.subcore_barrier()

        pltpu.sync_copy(acc.at[pl.ds(row0, ROWS_PER_TILE)],
                        out_hbm.at[c, pl.ds(row0, ROWS_PER_TILE)])

    return _agg_kernel


_agg128 = _make_agg(HIDDEN, CHUNK, SUPER, NBLK)
_agg16 = _make_agg(NUM_CLASSES, CHUNK_W, NCHUNK_W, 1)


def _norms_from_deg(degp_ref):
    deg_out = degp_ref[0, 0, :, 0] + degp_ref[1, 0, :, 0]
    deg_in = degp_ref[0, 1, :, 0] + degp_ref[1, 1, :, 0]
    norm_src = lax.rsqrt(jnp.maximum(deg_out, 1.0))
    norm_dst = lax.rsqrt(jnp.maximum(deg_in, 1.0))
    return norm_src, norm_dst


def _mm1_body(x_ref, degp_ref, w1_ref, o_ref):
    norm_src, _ = _norms_from_deg(degp_ref)
    h = x_ref[...] * norm_src[:, None]
    o_ref[...] = jnp.dot(h, w1_ref[...], preferred_element_type=jnp.float32)


def _mm1(x, degp, W1):
    return pl.pallas_call(
        _mm1_body,
        grid=(N_NODES // ROW_BLK,),
        in_specs=[
            pl.BlockSpec((ROW_BLK, IN_FEATS), lambda i: (i, 0)),
            pl.BlockSpec((NC, 2, ROW_BLK, 1), lambda i: (0, 0, i, 0)),
            pl.BlockSpec((IN_FEATS, HIDDEN), lambda i: (0, 0)),
        ],
        out_specs=pl.BlockSpec((ROW_BLK, HIDDEN), lambda i: (i, 0)),
        out_shape=jax.ShapeDtypeStruct((N_NODES, HIDDEN), jnp.float32),
    )(x, degp, W1)


def _mm2_body(aggp_ref, degp_ref, w2_ref, b1_ref, o_ref):
    norm_src, norm_dst = _norms_from_deg(degp_ref)
    agg = aggp_ref[0] + aggp_ref[1]
    out1 = jnp.maximum(agg * norm_dst[:, None] + b1_ref[...], 0.0)
    h2 = (out1 * norm_src[:, None])
    o_ref[...] = jnp.dot(h2, w2_ref[...], preferred_element_type=jnp.float32)


def _mm2(aggp, degp, W2, b1):
    return pl.pallas_call(
        _mm2_body,
        grid=(N_NODES // ROW_BLK,),
        in_specs=[
            pl.BlockSpec((NC, ROW_BLK, HIDDEN), lambda i: (0, i, 0)),
            pl.BlockSpec((NC, 2, ROW_BLK, 1), lambda i: (0, 0, i, 0)),
            pl.BlockSpec((HIDDEN, NUM_CLASSES), lambda i: (0, 0)),
            pl.BlockSpec((1, HIDDEN), lambda i: (0, 0)),
        ],
        out_specs=pl.BlockSpec((ROW_BLK, NUM_CLASSES), lambda i: (i, 0)),
        out_shape=jax.ShapeDtypeStruct((N_NODES, NUM_CLASSES), jnp.float32),
    )(aggp, degp, W2, b1)


def _fin_body(aggp_ref, degp_ref, b2_ref, o_ref):
    _, norm_dst = _norms_from_deg(degp_ref)
    agg = aggp_ref[0] + aggp_ref[1]
    o_ref[...] = agg * norm_dst[:, None] + b2_ref[...]


def _fin(aggp, degp, b2):
    return pl.pallas_call(
        _fin_body,
        grid=(N_NODES // ROW_BLK,),
        in_specs=[
            pl.BlockSpec((NC, ROW_BLK, NUM_CLASSES), lambda i: (0, i, 0)),
            pl.BlockSpec((NC, 2, ROW_BLK, 1), lambda i: (0, 0, i, 0)),
            pl.BlockSpec((1, NUM_CLASSES), lambda i: (0, 0)),
        ],
        out_specs=pl.BlockSpec((ROW_BLK, NUM_CLASSES), lambda i: (i, 0)),
        out_shape=jax.ShapeDtypeStruct((N_NODES, NUM_CLASSES), jnp.float32),
    )(aggp, degp, b2)


def kernel(inputs, edge_index, W1, b1, W2, b2):
    src = edge_index[0].astype(jnp.int32)
    dst = edge_index[1].astype(jnp.int32)
    shp128 = (NW, NBLK, SUPER, CHUNK)
    shp500 = (NW, 1, NCHUNK_W, CHUNK_W)
    shp_deg = (NW, NCHUNK_W, CHUNK_W)

    degp = _deg_kernel(src.reshape(shp_deg),
                       dst.reshape(shp_deg))
    degs = lax.slice(degp, (0, 0, 0, 0), (NC, 2, ACC_ROWS, 1))
    h1 = _mm1(inputs, degs, W1)
    aggp = _agg128(src.reshape(shp128), dst.reshape(shp128),
                   h1)
    h2 = _mm2(aggp, degs, W2, b1.reshape(1, -1))
    aggp2 = _agg16(src.reshape(shp500), dst.reshape(shp500),
                   h2)
    return _fin(aggp2, degs, b2.reshape(1, -1))


# speedup vs baseline: 1.0748x; 1.0748x over previous
import functools

import jax
import jax.numpy as jnp
from jax import lax
from jax.experimental import pallas as pl
from jax.experimental.pallas import tpu as pltpu
from jax.experimental.pallas import tpu_sc as plsc

N_NODES = 10000
N_EDGES = 320000
IN_FEATS = 128
HIDDEN = 128
NUM_CLASSES = 16

NC = 2
NS = 16
NW = NC * NS
EDGES_PER_W = 10240
E_PAD = NW * EDGES_PER_W
ACC_ROWS = 10240
ROWS_PER_TILE = ACC_ROWS // NS
ZROWS = 32
ROW_BLK = 2000

CHUNK = 128
SUPER = 16
NBLK = 5
CHUNK_W = 512
NCHUNK_W = EDGES_PER_W // CHUNK_W

_MESH = plsc.VectorSubcoreMesh(core_axis_name="c", subcore_axis_name="s")
_SC_PARAMS = pltpu.CompilerParams(use_tc_tiling_on_sc=False)


def _zero2d(ref, rows, cols):
    def body(r, carry):
        for k in range(cols // 16):
            ref[r, pl.ds(k * 16, 16)] = jnp.zeros((16,), jnp.float32)
        return carry
    lax.fori_loop(0, rows, body, 0)


def _fill_ones(ref, rows, cols):
    def body(r, carry):
        for k in range(cols // 16):
            ref[r, pl.ds(k * 16, 16)] = jnp.ones((16,), jnp.float32)
        return carry
    lax.fori_loop(0, rows, body, 0)


@functools.partial(
    pl.kernel,
    out_type=jax.ShapeDtypeStruct((NC, 2, ACC_ROWS, 16), jnp.float32),
    mesh=_MESH,
    compiler_params=_SC_PARAMS,
    scratch_types=[
        pltpu.VMEM((NCHUNK_W, CHUNK_W), jnp.int32),
        pltpu.VMEM((NCHUNK_W, CHUNK_W), jnp.int32),
        pltpu.VMEM((CHUNK_W, 16), jnp.float32),
        pltpu.VMEM((ZROWS, 16), jnp.float32),
        pltpu.VMEM_SHARED((ACC_ROWS, 16), jnp.float32),
        pltpu.VMEM_SHARED((ACC_ROWS, 16), jnp.float32),
    ],
)
def _deg_kernel(src_hbm, dst_hbm, out_hbm, src_v, dst_v, ones_v, zero_v,
                acc_src, acc_dst):
    c = lax.axis_index("c")
    s = lax.axis_index("s")
    wid = s * NC + c

    pltpu.sync_copy(src_hbm.at[wid], src_v)
    pltpu.sync_copy(dst_hbm.at[wid], dst_v)
    _fill_ones(ones_v, CHUNK_W, 16)
    _zero2d(zero_v, ZROWS, 16)
    row0 = s * ROWS_PER_TILE
    for t in range(ROWS_PER_TILE // ZROWS):
        pltpu.sync_copy(zero_v, acc_src.at[pl.ds(row0 + t * ZROWS, ZROWS)])
        pltpu.sync_copy(zero_v, acc_dst.at[pl.ds(row0 + t * ZROWS, ZROWS)])
    plsc.subcore_barrier()

    for j in range(NCHUNK_W):
        pltpu.sync_copy(ones_v, acc_src.at[src_v.at[j]], add=True)
        pltpu.sync_copy(ones_v, acc_dst.at[dst_v.at[j]], add=True)
    plsc.subcore_barrier()

    pltpu.sync_copy(acc_src.at[pl.ds(row0, ROWS_PER_TILE)],
                    out_hbm.at[c, 0, pl.ds(row0, ROWS_PER_TILE)])
    pltpu.sync_copy(acc_dst.at[pl.ds(row0, ROWS_PER_TILE)],
                    out_hbm.at[c, 1, pl.ds(row0, ROWS_PER_TILE)])


def _make_agg(F, ch, sup, nblk):
    @functools.partial(
        pl.kernel,
        out_type=jax.ShapeDtypeStruct((NC, ACC_ROWS, F), jnp.float32),
        mesh=_MESH,
        compiler_params=_SC_PARAMS,
        scratch_types=[
            pltpu.VMEM((sup, ch), jnp.int32),
            pltpu.VMEM((sup, ch), jnp.int32),
            pltpu.VMEM((ch, F), jnp.float32),
            pltpu.VMEM((ch, F), jnp.float32),
            pltpu.VMEM((ZROWS, F), jnp.float32),
            pltpu.VMEM_SHARED((ACC_ROWS, F), jnp.float32),
            pltpu.SemaphoreType.DMA,
            pltpu.SemaphoreType.DMA,
        ],
    )
    def _agg_kernel(src_hbm, dst_hbm, h_hbm, out_hbm, src_v, dst_v,
                    buf_a, buf_b, zero_v, acc, sem_a, sem_b):
        c = lax.axis_index("c")
        s = lax.axis_index("s")
        wid = s * NC + c

        _zero2d(zero_v, ZROWS, F)
        row0 = s * ROWS_PER_TILE
        for t in range(ROWS_PER_TILE // ZROWS):
            pltpu.sync_copy(zero_v, acc.at[pl.ds(row0 + t * ZROWS, ZROWS)])
        plsc.subcore_barrier()

        bufs = (buf_a, buf_b)
        sems = (sem_a, sem_b)
        for b in range(nblk):
            pltpu.sync_copy(src_hbm.at[wid, b], src_v)
            pltpu.sync_copy(dst_hbm.at[wid, b], dst_v)
            descs = [None, None]
            descs[0] = pltpu.async_copy(h_hbm.at[src_v.at[0]], bufs[0],
                                        sems[0])
            for j in range(sup):
                if j + 1 < sup:
                    descs[(j + 1) % 2] = pltpu.async_copy(
                        h_hbm.at[src_v.at[j + 1]], bufs[(j + 1) % 2],
                        sems[(j + 1) % 2])
                descs[j % 2].wait()
                pltpu.sync_copy(bufs[j % 2], acc.at[dst_v.at[j]], add=True)
        plsc.subcore_barrier()

        pltpu.sync_copy(acc.at[pl.ds(row0, ROWS_PER_TILE)],
                        out_hbm.at[c, pl.ds(row0, ROWS_PER_TILE)])

    return _agg_kernel


_agg128 = _make_agg(HIDDEN, CHUNK, SUPER, NBLK)
_agg16 = _make_agg(NUM_CLASSES, CHUNK_W, NCHUNK_W, 1)


def _norms_from_deg(degp_ref):
    deg_out = degp_ref[0, 0, :, 0] + degp_ref[1, 0, :, 0]
    deg_in = degp_ref[0, 1, :, 0] + degp_ref[1, 1, :, 0]
    norm_src = lax.rsqrt(jnp.maximum(deg_out, 1.0))
    norm_dst = lax.rsqrt(jnp.maximum(deg_in, 1.0))
    return norm_src, norm_dst


def _mm1_body(x_ref, degp_ref, w1_ref, o_ref):
    norm_src, _ = _norms_from_deg(degp_ref)
    h = x_ref[...] * norm_src[:, None]
    o_ref[...] = jnp.dot(h, w1_ref[...], preferred_element_type=jnp.float32)


def _mm1(x, degp, W1):
    return pl.pallas_call(
        _mm1_body,
        grid=(N_NODES // ROW_BLK,),
        in_specs=[
            pl.BlockSpec((ROW_BLK, IN_FEATS), lambda i: (i, 0)),
            pl.BlockSpec((NC, 2, ROW_BLK, 16), lambda i: (0, 0, i, 0)),
            pl.BlockSpec((IN_FEATS, HIDDEN), lambda i: (0, 0)),
        ],
        out_specs=pl.BlockSpec((ROW_BLK, HIDDEN), lambda i: (i, 0)),
        out_shape=jax.ShapeDtypeStruct((N_NODES, HIDDEN), jnp.float32),
    )(x, degp, W1)


def _mm2_body(aggp_ref, degp_ref, w2_ref, b1_ref, o_ref):
    norm_src, norm_dst = _norms_from_deg(degp_ref)
    agg = aggp_ref[0] + aggp_ref[1]
    out1 = jnp.maximum(agg * norm_dst[:, None] + b1_ref[...], 0.0)
    h2 = (out1 * norm_src[:, None])
    o_ref[...] = jnp.dot(h2, w2_ref[...], preferred_element_type=jnp.float32)


def _mm2(aggp, degp, W2, b1):
    return pl.pallas_call(
        _mm2_body,
        grid=(N_NODES // ROW_BLK,),
        in_specs=[
            pl.BlockSpec((NC, ROW_BLK, HIDDEN), lambda i: (0, i, 0)),
            pl.BlockSpec((NC, 2, ROW_BLK, 16), lambda i: (0, 0, i, 0)),
            pl.BlockSpec((HIDDEN, NUM_CLASSES), lambda i: (0, 0)),
            pl.BlockSpec((1, HIDDEN), lambda i: (0, 0)),
        ],
        out_specs=pl.BlockSpec((ROW_BLK, NUM_CLASSES), lambda i: (i, 0)),
        out_shape=jax.ShapeDtypeStruct((N_NODES, NUM_CLASSES), jnp.float32),
    )(aggp, degp, W2, b1)


PROWS = ACC_ROWS * NUM_CLASSES // 128
PBLK = 256


def _fin_body(aggp_ref, degin_ref, b2_ref, o_ref):
    deg_in = degin_ref[0, 0] + degin_ref[1, 0]
    norm_dst = lax.rsqrt(jnp.maximum(deg_in, 1.0))
    agg = aggp_ref[0] + aggp_ref[1]
    o_ref[...] = agg * norm_dst + b2_ref[...]


def _fin(aggp2, degp, b2):
    aggp2_p = aggp2.reshape(NC, ACC_ROWS * NUM_CLASSES // 128, 128)
    degp_p = degp.reshape(NC, 2, ACC_ROWS * 16 // 128, 128)
    b2t = jnp.tile(b2, 8).reshape(1, 128)
    out = pl.pallas_call(
        _fin_body,
        grid=(PROWS // PBLK,),
        in_specs=[
            pl.BlockSpec((NC, PBLK, 128), lambda i: (0, i, 0)),
            pl.BlockSpec((NC, 1, PBLK, 128), lambda i: (0, 1, i, 0)),
            pl.BlockSpec((1, 128), lambda i: (0, 0)),
        ],
        out_specs=pl.BlockSpec((PBLK, 128), lambda i: (i, 0)),
        out_shape=jax.ShapeDtypeStruct((PROWS, 128), jnp.float32),
    )(aggp2_p, degp_p, b2t)
    return out.reshape(ACC_ROWS, NUM_CLASSES)[:N_NODES]


def kernel(inputs, edge_index, W1, b1, W2, b2):
    n_pad = E_PAD - N_EDGES
    pad_pos = jnp.arange(n_pad, dtype=jnp.int32)
    pad_junk = N_NODES + pad_pos % (ACC_ROWS - N_NODES)
    pad_real = pad_pos % N_NODES
    src = edge_index[0].astype(jnp.int32)
    dst = edge_index[1].astype(jnp.int32)
    shp128 = (NW, NBLK, SUPER, CHUNK)
    shp512 = (NW, 1, NCHUNK_W, CHUNK_W)
    shp_deg = (NW, NCHUNK_W, CHUNK_W)
    src_all = jnp.concatenate([src, pad_real])
    src_junk = jnp.concatenate([src, pad_junk])
    dst_all = jnp.concatenate([dst, pad_junk])

    degp = _deg_kernel(src_junk.reshape(shp_deg),
                       dst_all.reshape(shp_deg))
    h1 = _mm1(inputs, degp, W1)
    aggp = _agg128(src_all.reshape(shp128), dst_all.reshape(shp128),
                   h1)
    h2 = _mm2(aggp, degp, W2, b1.reshape(1, -1))
    aggp2 = _agg16(src_all.reshape(shp512), dst_all.reshape(shp512),
                   h2)
    return _fin(aggp2, degp, b2)


# speedup vs baseline: 1.0762x; 1.0013x over previous
import functools

import jax
import jax.numpy as jnp
from jax import lax
from jax.experimental import pallas as pl
from jax.experimental.pallas import tpu as pltpu
from jax.experimental.pallas import tpu_sc as plsc

N_NODES = 10000
N_EDGES = 320000
IN_FEATS = 128
HIDDEN = 128
NUM_CLASSES = 16

NC = 2
NS = 16
NW = NC * NS
EDGES_PER_W = 10240
E_PAD = NW * EDGES_PER_W
ACC_ROWS = 10240
ROWS_PER_TILE = ACC_ROWS // NS
ZROWS = 32
ROW_BLK = 2000

CHUNK = 128
SUPER = 16
NBLK = 5
CHUNK_W = 512
NCHUNK_W = EDGES_PER_W // CHUNK_W

_MESH = plsc.VectorSubcoreMesh(core_axis_name="c", subcore_axis_name="s")
_SC_PARAMS = pltpu.CompilerParams(use_tc_tiling_on_sc=False)


def _zero2d(ref, rows, cols):
    def body(r, carry):
        for k in range(cols // 16):
            ref[r, pl.ds(k * 16, 16)] = jnp.zeros((16,), jnp.float32)
        return carry
    lax.fori_loop(0, rows, body, 0)


def _fill_ones(ref, rows, cols):
    def body(r, carry):
        for k in range(cols // 16):
            ref[r, pl.ds(k * 16, 16)] = jnp.ones((16,), jnp.float32)
        return carry
    lax.fori_loop(0, rows, body, 0)


@functools.partial(
    pl.kernel,
    out_type=jax.ShapeDtypeStruct((NC, 2, ACC_ROWS, 16), jnp.float32),
    mesh=_MESH,
    compiler_params=_SC_PARAMS,
    scratch_types=[
        pltpu.VMEM((NCHUNK_W, CHUNK_W), jnp.int32),
        pltpu.VMEM((NCHUNK_W, CHUNK_W), jnp.int32),
        pltpu.VMEM((CHUNK_W, 16), jnp.float32),
        pltpu.VMEM((ZROWS, 16), jnp.float32),
        pltpu.VMEM_SHARED((ACC_ROWS, 16), jnp.float32),
        pltpu.VMEM_SHARED((ACC_ROWS, 16), jnp.float32),
        pltpu.SemaphoreType.DMA,
        pltpu.SemaphoreType.DMA,
    ],
)
def _deg_kernel(src_hbm, dst_hbm, out_hbm, src_v, dst_v, ones_v, zero_v,
                acc_src, acc_dst, sem_s, sem_d):
    c = lax.axis_index("c")
    s = lax.axis_index("s")
    wid = s * NC + c

    pltpu.sync_copy(src_hbm.at[wid], src_v)
    pltpu.sync_copy(dst_hbm.at[wid], dst_v)
    _fill_ones(ones_v, CHUNK_W, 16)
    _zero2d(zero_v, ZROWS, 16)
    row0 = s * ROWS_PER_TILE
    for t in range(ROWS_PER_TILE // ZROWS):
        pltpu.sync_copy(zero_v, acc_src.at[pl.ds(row0 + t * ZROWS, ZROWS)])
        pltpu.sync_copy(zero_v, acc_dst.at[pl.ds(row0 + t * ZROWS, ZROWS)])
    plsc.subcore_barrier()

    descs = []
    for j in range(NCHUNK_W):
        descs.append(pltpu.async_copy(ones_v, acc_src.at[src_v.at[j]],
                                      sem_s, add=True))
        descs.append(pltpu.async_copy(ones_v, acc_dst.at[dst_v.at[j]],
                                      sem_d, add=True))
    for d in descs:
        d.wait()
    plsc.subcore_barrier()

    pltpu.sync_copy(acc_src.at[pl.ds(row0, ROWS_PER_TILE)],
                    out_hbm.at[c, 0, pl.ds(row0, ROWS_PER_TILE)])
    pltpu.sync_copy(acc_dst.at[pl.ds(row0, ROWS_PER_TILE)],
                    out_hbm.at[c, 1, pl.ds(row0, ROWS_PER_TILE)])


def _make_agg(F, ch, sup, nblk):
    @functools.partial(
        pl.kernel,
        out_type=jax.ShapeDtypeStruct((NC, ACC_ROWS, F), jnp.float32),
        mesh=_MESH,
        compiler_params=_SC_PARAMS,
        scratch_types=[
            pltpu.VMEM((sup, ch), jnp.int32),
            pltpu.VMEM((sup, ch), jnp.int32),
            pltpu.VMEM((ch, F), jnp.float32),
            pltpu.VMEM((ch, F), jnp.float32),
            pltpu.VMEM((ZROWS, F), jnp.float32),
            pltpu.VMEM_SHARED((ACC_ROWS, F), jnp.float32),
            pltpu.SemaphoreType.DMA,
            pltpu.SemaphoreType.DMA,
        ],
    )
    def _agg_kernel(src_hbm, dst_hbm, h_hbm, out_hbm, src_v, dst_v,
                    buf_a, buf_b, zero_v, acc, sem_a, sem_b):
        c = lax.axis_index("c")
        s = lax.axis_index("s")
        wid = s * NC + c

        _zero2d(zero_v, ZROWS, F)
        row0 = s * ROWS_PER_TILE
        for t in range(ROWS_PER_TILE // ZROWS):
            pltpu.sync_copy(zero_v, acc.at[pl.ds(row0 + t * ZROWS, ZROWS)])
        plsc.subcore_barrier()

        bufs = (buf_a, buf_b)
        sems = (sem_a, sem_b)
        for b in range(nblk):
            pltpu.sync_copy(src_hbm.at[wid, b], src_v)
            pltpu.sync_copy(dst_hbm.at[wid, b], dst_v)
            descs = [None, None]
            descs[0] = pltpu.async_copy(h_hbm.at[src_v.at[0]], bufs[0],
                                        sems[0])
            for j in range(sup):
                if j + 1 < sup:
                    descs[(j + 1) % 2] = pltpu.async_copy(
                        h_hbm.at[src_v.at[j + 1]], bufs[(j + 1) % 2],
                        sems[(j + 1) % 2])
                descs[j % 2].wait()
                pltpu.sync_copy(bufs[j % 2], acc.at[dst_v.at[j]], add=True)
        plsc.subcore_barrier()

        pltpu.sync_copy(acc.at[pl.ds(row0, ROWS_PER_TILE)],
                        out_hbm.at[c, pl.ds(row0, ROWS_PER_TILE)])

    return _agg_kernel


_agg128 = _make_agg(HIDDEN, CHUNK, SUPER, NBLK)
_agg16 = _make_agg(NUM_CLASSES, CHUNK_W, NCHUNK_W, 1)


def _norms_from_deg(degp_ref):
    deg_out = degp_ref[0, 0, :, 0] + degp_ref[1, 0, :, 0]
    deg_in = degp_ref[0, 1, :, 0] + degp_ref[1, 1, :, 0]
    norm_src = lax.rsqrt(jnp.maximum(deg_out, 1.0))
    norm_dst = lax.rsqrt(jnp.maximum(deg_in, 1.0))
    return norm_src, norm_dst


def _mm0_body(x_ref, w1_ref, o_ref):
    o_ref[...] = jnp.dot(x_ref[...], w1_ref[...],
                         preferred_element_type=jnp.float32)


def _mm0(x, W1):
    return pl.pallas_call(
        _mm0_body,
        grid=(N_NODES // ROW_BLK,),
        in_specs=[
            pl.BlockSpec((ROW_BLK, IN_FEATS), lambda i: (i, 0)),
            pl.BlockSpec((IN_FEATS, HIDDEN), lambda i: (0, 0)),
        ],
        out_specs=pl.BlockSpec((ROW_BLK, HIDDEN), lambda i: (i, 0)),
        out_shape=jax.ShapeDtypeStruct((N_NODES, HIDDEN), jnp.float32),
    )(x, W1)


def _scale_body(y_ref, degp_ref, o_ref):
    norm_src, _ = _norms_from_deg(degp_ref)
    o_ref[...] = y_ref[...] * norm_src[:, None]


def _scale(y, degp):
    return pl.pallas_call(
        _scale_body,
        grid=(N_NODES // ROW_BLK,),
        in_specs=[
            pl.BlockSpec((ROW_BLK, HIDDEN), lambda i: (i, 0)),
            pl.BlockSpec((NC, 2, ROW_BLK, 16), lambda i: (0, 0, i, 0)),
        ],
        out_specs=pl.BlockSpec((ROW_BLK, HIDDEN), lambda i: (i, 0)),
        out_shape=jax.ShapeDtypeStruct((N_NODES, HIDDEN), jnp.float32),
    )(y, degp)


def _mm2_body(aggp_ref, degp_ref, w2_ref, b1_ref, o_ref):
    norm_src, norm_dst = _norms_from_deg(degp_ref)
    agg = aggp_ref[0] + aggp_ref[1]
    out1 = jnp.maximum(agg * norm_dst[:, None] + b1_ref[...], 0.0)
    h2 = (out1 * norm_src[:, None])
    o_ref[...] = jnp.dot(h2, w2_ref[...], preferred_element_type=jnp.float32)


def _mm2(aggp, degp, W2, b1):
    return pl.pallas_call(
        _mm2_body,
        grid=(N_NODES // ROW_BLK,),
        in_specs=[
            pl.BlockSpec((NC, ROW_BLK, HIDDEN), lambda i: (0, i, 0)),
            pl.BlockSpec((NC, 2, ROW_BLK, 16), lambda i: (0, 0, i, 0)),
            pl.BlockSpec((HIDDEN, NUM_CLASSES), lambda i: (0, 0)),
            pl.BlockSpec((1, HIDDEN), lambda i: (0, 0)),
        ],
        out_specs=pl.BlockSpec((ROW_BLK, NUM_CLASSES), lambda i: (i, 0)),
        out_shape=jax.ShapeDtypeStruct((N_NODES, NUM_CLASSES), jnp.float32),
    )(aggp, degp, W2, b1)


PROWS = ACC_ROWS * NUM_CLASSES // 128
PBLK = 256


def _fin_body(aggp_ref, degin_ref, b2_ref, o_ref):
    deg_in = degin_ref[0, 0] + degin_ref[1, 0]
    norm_dst = lax.rsqrt(jnp.maximum(deg_in, 1.0))
    agg = aggp_ref[0] + aggp_ref[1]
    o_ref[...] = agg * norm_dst + b2_ref[...]


def _fin(aggp2, degp, b2):
    aggp2_p = aggp2.reshape(NC, ACC_ROWS * NUM_CLASSES // 128, 128)
    degp_p = degp.reshape(NC, 2, ACC_ROWS * 16 // 128, 128)
    b2t = jnp.tile(b2, 8).reshape(1, 128)
    out = pl.pallas_call(
        _fin_body,
        grid=(PROWS // PBLK,),
        in_specs=[
            pl.BlockSpec((NC, PBLK, 128), lambda i: (0, i, 0)),
            pl.BlockSpec((NC, 1, PBLK, 128), lambda i: (0, 1, i, 0)),
            pl.BlockSpec((1, 128), lambda i: (0, 0)),
        ],
        out_specs=pl.BlockSpec((PBLK, 128), lambda i: (i, 0)),
        out_shape=jax.ShapeDtypeStruct((PROWS, 128), jnp.float32),
    )(aggp2_p, degp_p, b2t)
    return out.reshape(ACC_ROWS, NUM_CLASSES)[:N_NODES]


def kernel(inputs, edge_index, W1, b1, W2, b2):
    n_pad = E_PAD - N_EDGES
    pad_pos = jnp.arange(n_pad, dtype=jnp.int32)
    pad_junk = N_NODES + pad_pos % (ACC_ROWS - N_NODES)
    pad_real = pad_pos % N_NODES
    src = edge_index[0].astype(jnp.int32)
    dst = edge_index[1].astype(jnp.int32)
    shp128 = (NW, NBLK, SUPER, CHUNK)
    shp512 = (NW, 1, NCHUNK_W, CHUNK_W)
    shp_deg = (NW, NCHUNK_W, CHUNK_W)
    src_all = jnp.concatenate([src, pad_real])
    src_junk = jnp.concatenate([src, pad_junk])
    dst_all = jnp.concatenate([dst, pad_junk])

    y = _mm0(inputs, W1)
    degp = _deg_kernel(src_junk.reshape(shp_deg),
                       dst_all.reshape(shp_deg))
    h1 = _scale(y, degp)
    aggp = _agg128(src_all.reshape(shp128), dst_all.reshape(shp128),
                   h1)
    h2 = _mm2(aggp, degp, W2, b1.reshape(1, -1))
    aggp2 = _agg16(src_all.reshape(shp512), dst_all.reshape(shp512),
                   h2)
    return _fin(aggp2, degp, b2)


# speedup vs baseline: 1.0847x; 1.0079x over previous
import functools

import jax
import jax.numpy as jnp
from jax import lax
from jax.experimental import pallas as pl
from jax.experimental.pallas import tpu as pltpu
from jax.experimental.pallas import tpu_sc as plsc

N_NODES = 10000
N_EDGES = 320000
IN_FEATS = 128
HIDDEN = 128
NUM_CLASSES = 16

NC = 2
NS = 16
NW = NC * NS
EDGES_PER_W = 10240
E_PAD = NW * EDGES_PER_W
ACC_ROWS = 10240
ROWS_PER_TILE = ACC_ROWS // NS
ZROWS = 32
ROW_BLK = 2000

CHUNK = 128
SUPER = 16
NBLK = 5
CHUNK_W = 1024
NCHUNK_W = EDGES_PER_W // CHUNK_W
ZROWS_W = 128

_MESH = plsc.VectorSubcoreMesh(core_axis_name="c", subcore_axis_name="s")
_SC_PARAMS = pltpu.CompilerParams(use_tc_tiling_on_sc=False)


def _zero2d(ref, rows, cols):
    def body(r, carry):
        for k in range(cols // 16):
            ref[r, pl.ds(k * 16, 16)] = jnp.zeros((16,), jnp.float32)
        return carry
    lax.fori_loop(0, rows, body, 0)


def _fill_ones(ref, rows, cols):
    def body(r, carry):
        for k in range(cols // 16):
            ref[r, pl.ds(k * 16, 16)] = jnp.ones((16,), jnp.float32)
        return carry
    lax.fori_loop(0, rows, body, 0)


@functools.partial(
    pl.kernel,
    out_type=jax.ShapeDtypeStruct((NC, 2, ACC_ROWS, 16), jnp.float32),
    mesh=_MESH,
    compiler_params=_SC_PARAMS,
    scratch_types=[
        pltpu.VMEM((NCHUNK_W, CHUNK_W), jnp.int32),
        pltpu.VMEM((NCHUNK_W, CHUNK_W), jnp.int32),
        pltpu.VMEM((CHUNK_W, 16), jnp.float32),
        pltpu.VMEM((ZROWS_W, 16), jnp.float32),
        pltpu.VMEM_SHARED((ACC_ROWS, 16), jnp.float32),
        pltpu.VMEM_SHARED((ACC_ROWS, 16), jnp.float32),
        pltpu.SemaphoreType.DMA,
        pltpu.SemaphoreType.DMA,
    ],
)
def _deg_kernel(src_hbm, dst_hbm, out_hbm, src_v, dst_v, ones_v, zero_v,
                acc_src, acc_dst, sem_s, sem_d):
    c = lax.axis_index("c")
    s = lax.axis_index("s")
    wid = s * NC + c

    pltpu.sync_copy(src_hbm.at[wid], src_v)
    pltpu.sync_copy(dst_hbm.at[wid], dst_v)
    _fill_ones(ones_v, CHUNK_W, 16)
    _zero2d(zero_v, ZROWS_W, 16)
    row0 = s * ROWS_PER_TILE
    for t in range(ROWS_PER_TILE // ZROWS_W):
        pltpu.sync_copy(zero_v, acc_src.at[pl.ds(row0 + t * ZROWS_W, ZROWS_W)])
        pltpu.sync_copy(zero_v, acc_dst.at[pl.ds(row0 + t * ZROWS_W, ZROWS_W)])
    plsc.subcore_barrier()

    descs = []
    for j in range(NCHUNK_W):
        descs.append(pltpu.async_copy(ones_v, acc_src.at[src_v.at[j]],
                                      sem_s, add=True))
        descs.append(pltpu.async_copy(ones_v, acc_dst.at[dst_v.at[j]],
                                      sem_d, add=True))
    for d in descs:
        d.wait()
    plsc.subcore_barrier()

    pltpu.sync_copy(acc_src.at[pl.ds(row0, ROWS_PER_TILE)],
                    out_hbm.at[c, 0, pl.ds(row0, ROWS_PER_TILE)])
    pltpu.sync_copy(acc_dst.at[pl.ds(row0, ROWS_PER_TILE)],
                    out_hbm.at[c, 1, pl.ds(row0, ROWS_PER_TILE)])


def _make_agg(F, ch, sup, nblk, zrows):
    @functools.partial(
        pl.kernel,
        out_type=jax.ShapeDtypeStruct((NC, ACC_ROWS, F), jnp.float32),
        mesh=_MESH,
        compiler_params=_SC_PARAMS,
        scratch_types=[
            pltpu.VMEM((sup, ch), jnp.int32),
            pltpu.VMEM((sup, ch), jnp.int32),
            pltpu.VMEM((ch, F), jnp.float32),
            pltpu.VMEM((ch, F), jnp.float32),
            pltpu.VMEM((zrows, F), jnp.float32),
            pltpu.VMEM_SHARED((ACC_ROWS, F), jnp.float32),
            pltpu.SemaphoreType.DMA,
            pltpu.SemaphoreType.DMA,
        ],
    )
    def _agg_kernel(src_hbm, dst_hbm, h_hbm, out_hbm, src_v, dst_v,
                    buf_a, buf_b, zero_v, acc, sem_a, sem_b):
        c = lax.axis_index("c")
        s = lax.axis_index("s")
        wid = s * NC + c

        _zero2d(zero_v, zrows, F)
        row0 = s * ROWS_PER_TILE
        for t in range(ROWS_PER_TILE // zrows):
            pltpu.sync_copy(zero_v, acc.at[pl.ds(row0 + t * zrows, zrows)])
        plsc.subcore_barrier()

        bufs = (buf_a, buf_b)
        sems = (sem_a, sem_b)
        for b in range(nblk):
            pltpu.sync_copy(src_hbm.at[wid, b], src_v)
            pltpu.sync_copy(dst_hbm.at[wid, b], dst_v)
            descs = [None, None]
            descs[0] = pltpu.async_copy(h_hbm.at[src_v.at[0]], bufs[0],
                                        sems[0])
            for j in range(sup):
                if j + 1 < sup:
                    descs[(j + 1) % 2] = pltpu.async_copy(
                        h_hbm.at[src_v.at[j + 1]], bufs[(j + 1) % 2],
                        sems[(j + 1) % 2])
                descs[j % 2].wait()
                pltpu.sync_copy(bufs[j % 2], acc.at[dst_v.at[j]], add=True)
        plsc.subcore_barrier()

        pltpu.sync_copy(acc.at[pl.ds(row0, ROWS_PER_TILE)],
                        out_hbm.at[c, pl.ds(row0, ROWS_PER_TILE)])

    return _agg_kernel


_agg128 = _make_agg(HIDDEN, CHUNK, SUPER, NBLK, ZROWS)
_agg16 = _make_agg(NUM_CLASSES, CHUNK_W, NCHUNK_W, 1, ZROWS_W)


def _norms_from_deg(degp_ref):
    deg_out = degp_ref[0, 0, :, 0] + degp_ref[1, 0, :, 0]
    deg_in = degp_ref[0, 1, :, 0] + degp_ref[1, 1, :, 0]
    norm_src = lax.rsqrt(jnp.maximum(deg_out, 1.0))
    norm_dst = lax.rsqrt(jnp.maximum(deg_in, 1.0))
    return norm_src, norm_dst


def _mm0_body(x_ref, w1_ref, o_ref):
    o_ref[...] = jnp.dot(x_ref[...], w1_ref[...],
                         preferred_element_type=jnp.float32)


def _mm0(x, W1):
    return pl.pallas_call(
        _mm0_body,
        grid=(N_NODES // ROW_BLK,),
        in_specs=[
            pl.BlockSpec((ROW_BLK, IN_FEATS), lambda i: (i, 0)),
            pl.BlockSpec((IN_FEATS, HIDDEN), lambda i: (0, 0)),
        ],
        out_specs=pl.BlockSpec((ROW_BLK, HIDDEN), lambda i: (i, 0)),
        out_shape=jax.ShapeDtypeStruct((N_NODES, HIDDEN), jnp.float32),
    )(x, W1)


def _scale_body(y_ref, degp_ref, o_ref):
    norm_src, _ = _norms_from_deg(degp_ref)
    o_ref[...] = y_ref[...] * norm_src[:, None]


def _scale(y, degp):
    return pl.pallas_call(
        _scale_body,
        grid=(N_NODES // ROW_BLK,),
        in_specs=[
            pl.BlockSpec((ROW_BLK, HIDDEN), lambda i: (i, 0)),
            pl.BlockSpec((NC, 2, ROW_BLK, 16), lambda i: (0, 0, i, 0)),
        ],
        out_specs=pl.BlockSpec((ROW_BLK, HIDDEN), lambda i: (i, 0)),
        out_shape=jax.ShapeDtypeStruct((N_NODES, HIDDEN), jnp.float32),
    )(y, degp)


def _mm2_body(aggp_ref, degp_ref, w2_ref, b1_ref, o_ref):
    norm_src, norm_dst = _norms_from_deg(degp_ref)
    agg = aggp_ref[0] + aggp_ref[1]
    out1 = jnp.maximum(agg * norm_dst[:, None] + b1_ref[...], 0.0)
    h2 = (out1 * norm_src[:, None])
    o_ref[...] = jnp.dot(h2, w2_ref[...], preferred_element_type=jnp.float32)


def _mm2(aggp, degp, W2, b1):
    return pl.pallas_call(
        _mm2_body,
        grid=(N_NODES // ROW_BLK,),
        in_specs=[
            pl.BlockSpec((NC, ROW_BLK, HIDDEN), lambda i: (0, i, 0)),
            pl.BlockSpec((NC, 2, ROW_BLK, 16), lambda i: (0, 0, i, 0)),
            pl.BlockSpec((HIDDEN, NUM_CLASSES), lambda i: (0, 0)),
            pl.BlockSpec((1, HIDDEN), lambda i: (0, 0)),
        ],
        out_specs=pl.BlockSpec((ROW_BLK, NUM_CLASSES), lambda i: (i, 0)),
        out_shape=jax.ShapeDtypeStruct((N_NODES, NUM_CLASSES), jnp.float32),
    )(aggp, degp, W2, b1)


PROWS = ACC_ROWS * NUM_CLASSES // 128
PBLK = 256


def _fin_body(aggp_ref, degin_ref, b2_ref, o_ref):
    deg_in = degin_ref[0, 0] + degin_ref[1, 0]
    norm_dst = lax.rsqrt(jnp.maximum(deg_in, 1.0))
    agg = aggp_ref[0] + aggp_ref[1]
    o_ref[...] = agg * norm_dst + b2_ref[...]


def _fin(aggp2, degp, b2):
    aggp2_p = aggp2.reshape(NC, ACC_ROWS * NUM_CLASSES // 128, 128)
    degp_p = degp.reshape(NC, 2, ACC_ROWS * 16 // 128, 128)
    b2t = jnp.tile(b2, 8).reshape(1, 128)
    out = pl.pallas_call(
        _fin_body,
        grid=(PROWS // PBLK,),
        in_specs=[
            pl.BlockSpec((NC, PBLK, 128), lambda i: (0, i, 0)),
            pl.BlockSpec((NC, 1, PBLK, 128), lambda i: (0, 1, i, 0)),
            pl.BlockSpec((1, 128), lambda i: (0, 0)),
        ],
        out_specs=pl.BlockSpec((PBLK, 128), lambda i: (i, 0)),
        out_shape=jax.ShapeDtypeStruct((PROWS, 128), jnp.float32),
    )(aggp2_p, degp_p, b2t)
    return out.reshape(ACC_ROWS, NUM_CLASSES)[:N_NODES]


def kernel(inputs, edge_index, W1, b1, W2, b2):
    n_pad = E_PAD - N_EDGES
    pad_pos = jnp.arange(n_pad, dtype=jnp.int32)
    pad_junk = N_NODES + pad_pos % (ACC_ROWS - N_NODES)
    pad_real = pad_pos % N_NODES
    src = edge_index[0].astype(jnp.int32)
    dst = edge_index[1].astype(jnp.int32)
    shp128 = (NW, NBLK, SUPER, CHUNK)
    shp512 = (NW, 1, NCHUNK_W, CHUNK_W)
    shp_deg = (NW, NCHUNK_W, CHUNK_W)
    src_all = jnp.concatenate([src, pad_real])
    src_junk = jnp.concatenate([src, pad_junk])
    dst_all = jnp.concatenate([dst, pad_junk])

    y = _mm0(inputs, W1)
    degp = _deg_kernel(src_junk.reshape(shp_deg),
                       dst_all.reshape(shp_deg))
    h1 = _scale(y, degp)
    aggp = _agg128(src_all.reshape(shp128), dst_all.reshape(shp128),
                   h1)
    h2 = _mm2(aggp, degp, W2, b1.reshape(1, -1))
    aggp2 = _agg16(src_all.reshape(shp512), dst_all.reshape(shp512),
                   h2)
    return _fin(aggp2, degp, b2)


# speedup vs baseline: 1.0850x; 1.0004x over previous
import functools

import jax
import jax.numpy as jnp
from jax import lax
from jax.experimental import pallas as pl
from jax.experimental.pallas import tpu as pltpu
from jax.experimental.pallas import tpu_sc as plsc

N_NODES = 10000
N_EDGES = 320000
IN_FEATS = 128
HIDDEN = 128
NUM_CLASSES = 16

NC = 2
NS = 16
NW = NC * NS
EDGES_PER_W = 10240
E_PAD = NW * EDGES_PER_W
ACC_ROWS = 10240
ROWS_PER_TILE = ACC_ROWS // NS
ZROWS = 32
ROW_BLK = 2000

CHUNK = 128
SUPER = 16
NBLK = 5
CHUNK_W = 1024
NCHUNK_W = EDGES_PER_W // CHUNK_W
ZROWS_W = 128

_MESH = plsc.VectorSubcoreMesh(core_axis_name="c", subcore_axis_name="s")
_SC_PARAMS = pltpu.CompilerParams(use_tc_tiling_on_sc=False)


def _zero2d(ref, rows, cols):
    def body(r, carry):
        for k in range(cols // 16):
            ref[r, pl.ds(k * 16, 16)] = jnp.zeros((16,), jnp.float32)
        return carry
    lax.fori_loop(0, rows, body, 0)


def _fill_ones(ref, rows, cols):
    def body(r, carry):
        for k in range(cols // 16):
            ref[r, pl.ds(k * 16, 16)] = jnp.ones((16,), jnp.float32)
        return carry
    lax.fori_loop(0, rows, body, 0)


@functools.partial(
    pl.kernel,
    out_type=jax.ShapeDtypeStruct((NC, 2, ACC_ROWS, 16), jnp.float32),
    mesh=_MESH,
    compiler_params=_SC_PARAMS,
    scratch_types=[
        pltpu.VMEM((NCHUNK_W, CHUNK_W), jnp.int32),
        pltpu.VMEM((NCHUNK_W, CHUNK_W), jnp.int32),
        pltpu.VMEM((CHUNK_W, 16), jnp.float32),
        pltpu.VMEM((ZROWS_W, 16), jnp.float32),
        pltpu.VMEM_SHARED((ACC_ROWS, 16), jnp.float32),
        pltpu.VMEM_SHARED((ACC_ROWS, 16), jnp.float32),
        pltpu.SemaphoreType.DMA,
        pltpu.SemaphoreType.DMA,
    ],
)
def _deg_kernel(src_hbm, dst_hbm, out_hbm, src_v, dst_v, ones_v, zero_v,
                acc_src, acc_dst, sem_s, sem_d):
    c = lax.axis_index("c")
    s = lax.axis_index("s")
    wid = s * NC + c

    pltpu.sync_copy(src_hbm.at[wid], src_v)
    pltpu.sync_copy(dst_hbm.at[wid], dst_v)
    _fill_ones(ones_v, CHUNK_W, 16)
    _zero2d(zero_v, ZROWS_W, 16)
    row0 = s * ROWS_PER_TILE
    for t in range(ROWS_PER_TILE // ZROWS_W):
        pltpu.sync_copy(zero_v, acc_src.at[pl.ds(row0 + t * ZROWS_W, ZROWS_W)])
        pltpu.sync_copy(zero_v, acc_dst.at[pl.ds(row0 + t * ZROWS_W, ZROWS_W)])
    plsc.subcore_barrier()

    descs = []
    for j in range(NCHUNK_W):
        descs.append(pltpu.async_copy(ones_v, acc_src.at[src_v.at[j]],
                                      sem_s, add=True))
        descs.append(pltpu.async_copy(ones_v, acc_dst.at[dst_v.at[j]],
                                      sem_d, add=True))
    for d in descs:
        d.wait()
    plsc.subcore_barrier()

    pltpu.sync_copy(acc_src.at[pl.ds(row0, ROWS_PER_TILE)],
                    out_hbm.at[c, 0, pl.ds(row0, ROWS_PER_TILE)])
    pltpu.sync_copy(acc_dst.at[pl.ds(row0, ROWS_PER_TILE)],
                    out_hbm.at[c, 1, pl.ds(row0, ROWS_PER_TILE)])


def _make_agg(F, ch, sup, nblk, zrows):
    @functools.partial(
        pl.kernel,
        out_type=jax.ShapeDtypeStruct((NC, ACC_ROWS, F), jnp.float32),
        mesh=_MESH,
        compiler_params=_SC_PARAMS,
        scratch_types=[
            pltpu.VMEM((sup, ch), jnp.int32),
            pltpu.VMEM((sup, ch), jnp.int32),
            pltpu.VMEM((ch, F), jnp.float32),
            pltpu.VMEM((ch, F), jnp.float32),
            pltpu.VMEM((zrows, F), jnp.float32),
            pltpu.VMEM_SHARED((ACC_ROWS, F), jnp.float32),
            pltpu.SemaphoreType.DMA,
            pltpu.SemaphoreType.DMA,
            pltpu.SemaphoreType.DMA,
            pltpu.SemaphoreType.DMA,
        ],
    )
    def _agg_kernel(src_hbm, dst_hbm, h_hbm, out_hbm, src_v, dst_v,
                    buf_a, buf_b, zero_v, acc, sem_a, sem_b, sem_c, sem_d):
        c = lax.axis_index("c")
        s = lax.axis_index("s")
        wid = s * NC + c

        _zero2d(zero_v, zrows, F)
        row0 = s * ROWS_PER_TILE
        for t in range(ROWS_PER_TILE // zrows):
            pltpu.sync_copy(zero_v, acc.at[pl.ds(row0 + t * zrows, zrows)])
        plsc.subcore_barrier()

        bufs = (buf_a, buf_b)
        gsems = (sem_a, sem_b)
        ssems = (sem_c, sem_d)
        for b in range(nblk):
            pltpu.sync_copy(src_hbm.at[wid, b], src_v)
            pltpu.sync_copy(dst_hbm.at[wid, b], dst_v)
            gdescs = [None, None]
            sdescs = [None, None]
            gdescs[0] = pltpu.async_copy(h_hbm.at[src_v.at[0]], bufs[0],
                                         gsems[0])
            for j in range(sup):
                k = j % 2
                n = (j + 1) % 2
                if j + 1 < sup:
                    if sdescs[n] is not None:
                        sdescs[n].wait()
                        sdescs[n] = None
                    gdescs[n] = pltpu.async_copy(
                        h_hbm.at[src_v.at[j + 1]], bufs[n], gsems[n])
                gdescs[k].wait()
                sdescs[k] = pltpu.async_copy(bufs[k], acc.at[dst_v.at[j]],
                                             ssems[k], add=True)
            for d in sdescs:
                if d is not None:
                    d.wait()
        plsc.subcore_barrier()

        pltpu.sync_copy(acc.at[pl.ds(row0, ROWS_PER_TILE)],
                        out_hbm.at[c, pl.ds(row0, ROWS_PER_TILE)])

    return _agg_kernel


_agg128 = _make_agg(HIDDEN, CHUNK, SUPER, NBLK, ZROWS)
_agg16 = _make_agg(NUM_CLASSES, CHUNK_W, NCHUNK_W, 1, ZROWS_W)


def _norms_from_deg(degp_ref):
    deg_out = degp_ref[0, 0, :, 0] + degp_ref[1, 0, :, 0]
    deg_in = degp_ref[0, 1, :, 0] + degp_ref[1, 1, :, 0]
    norm_src = lax.rsqrt(jnp.maximum(deg_out, 1.0))
    norm_dst = lax.rsqrt(jnp.maximum(deg_in, 1.0))
    return norm_src, norm_dst


def _mm0_body(x_ref, w1_ref, o_ref):
    o_ref[...] = jnp.dot(x_ref[...], w1_ref[...],
                         preferred_element_type=jnp.float32)


def _mm0(x, W1):
    return pl.pallas_call(
        _mm0_body,
        grid=(N_NODES // ROW_BLK,),
        in_specs=[
            pl.BlockSpec((ROW_BLK, IN_FEATS), lambda i: (i, 0)),
            pl.BlockSpec((IN_FEATS, HIDDEN), lambda i: (0, 0)),
        ],
        out_specs=pl.BlockSpec((ROW_BLK, HIDDEN), lambda i: (i, 0)),
        out_shape=jax.ShapeDtypeStruct((N_NODES, HIDDEN), jnp.float32),
    )(x, W1)


def _scale_body(y_ref, degp_ref, o_ref):
    norm_src, _ = _norms_from_deg(degp_ref)
    o_ref[...] = y_ref[...] * norm_src[:, None]


def _scale(y, degp):
    return pl.pallas_call(
        _scale_body,
        grid=(N_NODES // ROW_BLK,),
        in_specs=[
            pl.BlockSpec((ROW_BLK, HIDDEN), lambda i: (i, 0)),
            pl.BlockSpec((NC, 2, ROW_BLK, 16), lambda i: (0, 0, i, 0)),
        ],
        out_specs=pl.BlockSpec((ROW_BLK, HIDDEN), lambda i: (i, 0)),
        out_shape=jax.ShapeDtypeStruct((N_NODES, HIDDEN), jnp.float32),
    )(y, degp)


def _mm2_body(aggp_ref, degp_ref, w2_ref, b1_ref, o_ref):
    norm_src, norm_dst = _norms_from_deg(degp_ref)
    agg = aggp_ref[0] + aggp_ref[1]
    out1 = jnp.maximum(agg * norm_dst[:, None] + b1_ref[...], 0.0)
    h2 = (out1 * norm_src[:, None])
    o_ref[...] = jnp.dot(h2, w2_ref[...], preferred_element_type=jnp.float32)


def _mm2(aggp, degp, W2, b1):
    return pl.pallas_call(
        _mm2_body,
        grid=(N_NODES // ROW_BLK,),
        in_specs=[
            pl.BlockSpec((NC, ROW_BLK, HIDDEN), lambda i: (0, i, 0)),
            pl.BlockSpec((NC, 2, ROW_BLK, 16), lambda i: (0, 0, i, 0)),
            pl.BlockSpec((HIDDEN, NUM_CLASSES), lambda i: (0, 0)),
            pl.BlockSpec((1, HIDDEN), lambda i: (0, 0)),
        ],
        out_specs=pl.BlockSpec((ROW_BLK, NUM_CLASSES), lambda i: (i, 0)),
        out_shape=jax.ShapeDtypeStruct((N_NODES, NUM_CLASSES), jnp.float32),
    )(aggp, degp, W2, b1)


PROWS = ACC_ROWS * NUM_CLASSES // 128
PBLK = 256


def _fin_body(aggp_ref, degin_ref, b2_ref, o_ref):
    deg_in = degin_ref[0, 0] + degin_ref[1, 0]
    norm_dst = lax.rsqrt(jnp.maximum(deg_in, 1.0))
    agg = aggp_ref[0] + aggp_ref[1]
    o_ref[...] = agg * norm_dst + b2_ref[...]


def _fin(aggp2, degp, b2):
    aggp2_p = aggp2.reshape(NC, ACC_ROWS * NUM_CLASSES // 128, 128)
    degp_p = degp.reshape(NC, 2, ACC_ROWS * 16 // 128, 128)
    b2t = jnp.tile(b2, 8).reshape(1, 128)
    out = pl.pallas_call(
        _fin_body,
        grid=(PROWS // PBLK,),
        in_specs=[
            pl.BlockSpec((NC, PBLK, 128), lambda i: (0, i, 0)),
            pl.BlockSpec((NC, 1, PBLK, 128), lambda i: (0, 1, i, 0)),
            pl.BlockSpec((1, 128), lambda i: (0, 0)),
        ],
        out_specs=pl.BlockSpec((PBLK, 128), lambda i: (i, 0)),
        out_shape=jax.ShapeDtypeStruct((PROWS, 128), jnp.float32),
    )(aggp2_p, degp_p, b2t)
    return out.reshape(ACC_ROWS, NUM_CLASSES)[:N_NODES]


def kernel(inputs, edge_index, W1, b1, W2, b2):
    n_pad = E_PAD - N_EDGES
    pad_pos = jnp.arange(n_pad, dtype=jnp.int32)
    pad_junk = N_NODES + pad_pos % (ACC_ROWS - N_NODES)
    pad_real = pad_pos % N_NODES
    src = edge_index[0].astype(jnp.int32)
    dst = edge_index[1].astype(jnp.int32)
    shp128 = (NW, NBLK, SUPER, CHUNK)
    shp512 = (NW, 1, NCHUNK_W, CHUNK_W)
    shp_deg = (NW, NCHUNK_W, CHUNK_W)
    src_all = jnp.concatenate([src, pad_real])
    src_junk = jnp.concatenate([src, pad_junk])
    dst_all = jnp.concatenate([dst, pad_junk])

    y = _mm0(inputs, W1)
    degp = _deg_kernel(src_junk.reshape(shp_deg),
                       dst_all.reshape(shp_deg))
    h1 = _scale(y, degp)
    aggp = _agg128(src_all.reshape(shp128), dst_all.reshape(shp128),
                   h1)
    h2 = _mm2(aggp, degp, W2, b1.reshape(1, -1))
    aggp2 = _agg16(src_all.reshape(shp512), dst_all.reshape(shp512),
                   h2)
    return _fin(aggp2, degp, b2)
